# baseline (speedup 1.0000x reference)
import jax
import jax.numpy as jnp
from jax import lax
from jax.experimental import pallas as pl
from jax.experimental.pallas import tpu as pltpu

_BN_EPS = 1e-5
_C = 3

_PAIRS = ((0, 0), (0, 1), (0, 2), (1, 1), (1, 2), (2, 2))
_NSTAT = _C + len(_PAIRS)
_LANE = 128
_SUB = 8
_NPARAM = 2 * _C + 3
_TILE_TARGET = 3 * 1024 * 1024
_IN_SLOTS = 6
_OUT_SLOTS = 3


def _round_up(v, m):
    return -(-v // m) * m


def _part_sum(a):
    lane = a.shape[-1]
    rows = a.size // lane
    if rows % _SUB == 0:
        return a.reshape(rows // _SUB, _SUB, lane).sum(axis=0)
    return a.reshape(rows, lane).sum(axis=0, keepdims=True)


def _tile_stats(xv):
    xs = [xv[:, c:c + 1, :, :] for c in range(_C)]
    parts = [_part_sum(xs[c]) for c in range(_C)]
    parts += [_part_sum(xs[i] * xs[j]) for (i, j) in _PAIRS]
    return parts


def _fold_affine(tot, p_ref, inv_m):
    mean = [tot[c] * inv_m for c in range(_C)]
    exx = {}
    for k, (i, j) in enumerate(_PAIRS):
        exx[(i, j)] = tot[_C + k] * inv_m
        exx[(j, i)] = exx[(i, j)]
    cov = [[exx[(i, j)] - mean[i] * mean[j] for j in range(_C)]
           for i in range(_C)]
    w1s = [[p_ref[i, j] for j in range(_C)] for i in range(_C)]
    w2s = [[p_ref[i, _C + j] for j in range(_C)] for i in range(_C)]
    g = []
    for c in range(_C):
        vh = sum(w1s[c][i] * cov[i][j] * w1s[c][j]
                 for i in range(_C) for j in range(_C))
        vh = jnp.maximum(vh, 0.0) + _BN_EPS
        rs = lax.rsqrt(jnp.full((1, _LANE), vh, jnp.float32))[0, 0]
        g.append(p_ref[c, 2 * _C] * rs)
    mh = [sum(w1s[k][i] * mean[i] for i in range(_C)) for k in range(_C)]
    w_eff = [[sum(w2s[c][k] * g[k] * w1s[k][j] for k in range(_C))
              for j in range(_C)] for c in range(_C)]
    b_eff = [p_ref[c, 2 * _C + 2]
             + sum(w2s[c][k] * (p_ref[k, 2 * _C + 1] - g[k] * mh[k])
                   for k in range(_C))
             for c in range(_C)]
    return w_eff, b_eff


def _apply_affine(xv, w_eff, b_eff):
    xs = [xv[:, c:c + 1, :, :] for c in range(_C)]
    return jnp.concatenate(
        [w_eff[c][0] * xs[0] + w_eff[c][1] * xs[1]
         + w_eff[c][2] * xs[2] + b_eff[c] for c in range(_C)], axis=1)


def _forward_native(x_nchw, params, inv_m, Nb):
    N, _, H, W = x_nchw.shape
    T = N // Nb
    n_in = min(_IN_SLOTS, T)
    n_out = min(_OUT_SLOTS, T)

    def body(p_ref, x_hbm, o_hbm, ibuf, obuf, in_sems, out_sems):
        def in_copy(t):
            return pltpu.make_async_copy(
                x_hbm.at[pl.ds(t * Nb, Nb)], ibuf.at[t % n_in],
                in_sems.at[t % n_in])

        def out_copy(t):
            return pltpu.make_async_copy(
                obuf.at[t % n_out], o_hbm.at[pl.ds(t * Nb, Nb)],
                out_sems.at[t % n_out])

        for t in range(n_in):
            in_copy(t).start()
        tot9 = None
        for t in range(T):
            in_copy(t).wait()
            parts = _tile_stats(ibuf[t % n_in])
            tot9 = parts if tot9 is None else [
                a + b for a, b in zip(tot9, parts)]
            if t + n_in < T:
                in_copy(t + n_in).start()

        tot = [jnp.sum(v) for v in tot9]
        w_eff, b_eff = _fold_affine(tot, p_ref, inv_m)

        for t in range(n_in):
            in_copy(t).start()
        for t in range(T):
            in_copy(t).wait()
            if t >= n_out:
                out_copy(t - n_out).wait()
            obuf[t % n_out] = _apply_affine(ibuf[t % n_in], w_eff, b_eff)
            out_copy(t).start()
            if t + n_in < T:
                in_copy(t + n_in).start()
        for t in range(max(0, T - n_out), T):
            out_copy(t).wait()

    return pl.pallas_call(
        body,
        out_shape=jax.ShapeDtypeStruct((N, _C, H, W), jnp.float32),
        in_specs=[pl.BlockSpec(memory_space=pltpu.MemorySpace.SMEM),
                  pl.BlockSpec(memory_space=pl.MemorySpace.ANY)],
        out_specs=pl.BlockSpec(memory_space=pl.MemorySpace.ANY),
        scratch_shapes=[
            pltpu.VMEM((n_in, Nb, _C, H, W), jnp.float32),
            pltpu.VMEM((n_out, Nb, _C, H, W), jnp.float32),
            pltpu.SemaphoreType.DMA((n_in,)),
            pltpu.SemaphoreType.DMA((n_out,)),
        ],
        compiler_params=pltpu.CompilerParams(
            vmem_limit_bytes=64 * 1024 * 1024),
    )(params, x_nchw)


def _plan_tiles(rows, n):
    per_sample = _C * rows * _LANE * 4
    if per_sample <= _TILE_TARGET:
        s = rows
        nb = 1
        want = max(1, _TILE_TARGET // per_sample)
        for d in range(1, n + 1):
            if n % d == 0 and d <= want:
                nb = d
    else:
        nb = 1
        s = _SUB
        cap = _TILE_TARGET // (_C * _LANE * 4)
        for cand in range(_SUB, rows + 1, _SUB):
            if rows % cand == 0 and cand <= cap:
                s = cand
    return nb, s


def _forward_streaming(x_nchw, params, inv_m):
    N, _, H, W = x_nchw.shape
    HW = H * W
    HWp = _round_up(HW, _LANE * _SUB)
    ROWS = HWp // _LANE
    x3 = x_nchw.reshape(N, _C, HW)
    if HWp != HW:
        x3 = jnp.pad(x3, ((0, 0), (0, 0), (0, HWp - HW)))
    x4 = x3.reshape(N, _C, ROWS, _LANE)
    Nb, S = _plan_tiles(ROWS, N)
    tn = N // Nb
    tr = ROWS // S
    T = tn * tr

    def body(p_ref, x_ref, o_ref, acc_ref, wb_ref):
        ph = pl.program_id(0)
        t = pl.program_id(1) * tr + pl.program_id(2)

        @pl.when(jnp.logical_and(ph == 0, t == 0))
        def _init():
            acc_ref[...] = jnp.zeros_like(acc_ref)

        @pl.when(ph == 0)
        def _stats():
            acc_ref[...] += jnp.stack(_tile_stats(x_ref[...]), axis=0)

        @pl.when(jnp.logical_and(ph == 0, t == T - 1))
        def _fold():
            tot = [jnp.sum(acc_ref[k]) for k in range(_NSTAT)]
            w_eff, b_eff = _fold_affine(tot, p_ref, inv_m)
            for c in range(_C):
                for j in range(_C):
                    wb_ref[c, j] = w_eff[c][j]
                wb_ref[c, _C] = b_eff[c]

        @pl.when(ph == 1)
        def _apply():
            w_eff = [[wb_ref[c, j] for j in range(_C)] for c in range(_C)]
            b_eff = [wb_ref[c, _C] for c in range(_C)]
            o_ref[...] = _apply_affine(x_ref[...], w_eff, b_eff)

    x_spec = pl.BlockSpec((Nb, _C, S, _LANE), lambda p, n, r: (n, 0, r, 0))
    o_spec = pl.BlockSpec(
        (Nb, _C, S, _LANE),
        lambda p, n, r: (jnp.where(p == 0, 0, n), 0,
                         jnp.where(p == 0, 0, r), 0))
    p_spec = pl.BlockSpec((_C, _NPARAM), lambda p, n, r: (0, 0),
                          memory_space=pltpu.MemorySpace.SMEM)

    out4 = pl.pallas_call(
        body,
        out_shape=jax.ShapeDtypeStruct((N, _C, ROWS, _LANE), jnp.float32),
        grid=(2, tn, tr),
        in_specs=[p_spec, x_spec],
        out_specs=o_spec,
        scratch_shapes=[pltpu.VMEM((_NSTAT, _SUB, _LANE), jnp.float32),
                        pltpu.SMEM((_C, _C + 1), jnp.float32)],
        compiler_params=pltpu.CompilerParams(
            dimension_semantics=("arbitrary", "arbitrary", "arbitrary"),
            vmem_limit_bytes=64 * 1024 * 1024),
    )(params, x4)

    out3 = out4.reshape(N, _C, HWp)
    if HWp != HW:
        out3 = out3[:, :, :HW]
    return out3.reshape(N, _C, H, W)


def _fused_forward(x_nchw, w1, b1, gamma, beta, w2, b2):
    del b1
    N, c_in, H, W = x_nchw.shape
    assert c_in == _C
    x_nchw = x_nchw.astype(jnp.float32)
    inv_m = 1.0 / float(N * H * W)

    params = jnp.concatenate(
        [w1.astype(jnp.float32), w2.astype(jnp.float32),
         gamma.astype(jnp.float32)[:, None],
         beta.astype(jnp.float32)[:, None],
         b2.astype(jnp.float32)[:, None]], axis=1)

    tile1 = _C * _round_up(H, _SUB) * _round_up(W, _LANE) * 4
    if tile1 <= _TILE_TARGET:
        nb = 1
        want = max(1, _TILE_TARGET // tile1)
        for d in range(1, N + 1):
            if N % d == 0 and d <= want:
                nb = d
        return _forward_native(x_nchw, params, inv_m, nb)
    return _forward_streaming(x_nchw, params, inv_m)


def kernel(x_nchw, w1, b1, gamma, beta, w2, b2):
    return _fused_forward(x_nchw, w1, b1, gamma, beta, w2, b2)


# speedup vs baseline: 1.8994x; 1.8994x over previous
import jax
import jax.numpy as jnp
from jax import lax
from jax.experimental import pallas as pl
from jax.experimental.pallas import tpu as pltpu

_BN_EPS = 1e-5
_C = 3

_PAIRS = ((0, 0), (0, 1), (0, 2), (1, 1), (1, 2), (2, 2))
_NSTAT = _C + len(_PAIRS)
_LANE = 128
_SUB = 8
_NPARAM = 2 * _C + 3
_TILE_TARGET = 2 * 1024 * 1024
_RESIDENT_LIMIT = 40 * 1024 * 1024
_OUT_SLOTS = 3


def _round_up(v, m):
    return -(-v // m) * m


def _part_sum(a):
    lane = a.shape[-1]
    rows = a.size // lane
    if rows % _SUB == 0:
        return a.reshape(rows // _SUB, _SUB, lane).sum(axis=0)
    return a.reshape(rows, lane).sum(axis=0, keepdims=True)


def _tile_stats(xv):
    xs = [xv[:, c:c + 1, :, :] for c in range(_C)]
    parts = [_part_sum(xs[c]) for c in range(_C)]
    parts += [_part_sum(xs[i] * xs[j]) for (i, j) in _PAIRS]
    return parts


def _fold_affine(tot, p_ref, inv_m):
    mean = [tot[c] * inv_m for c in range(_C)]
    exx = {}
    for k, (i, j) in enumerate(_PAIRS):
        exx[(i, j)] = tot[_C + k] * inv_m
        exx[(j, i)] = exx[(i, j)]
    cov = [[exx[(i, j)] - mean[i] * mean[j] for j in range(_C)]
           for i in range(_C)]
    w1s = [[p_ref[i, j] for j in range(_C)] for i in range(_C)]
    w2s = [[p_ref[i, _C + j] for j in range(_C)] for i in range(_C)]
    g = []
    for c in range(_C):
        vh = sum(w1s[c][i] * cov[i][j] * w1s[c][j]
                 for i in range(_C) for j in range(_C))
        vh = jnp.maximum(vh, 0.0) + _BN_EPS
        rs = lax.rsqrt(jnp.full((1, _LANE), vh, jnp.float32))[0, 0]
        g.append(p_ref[c, 2 * _C] * rs)
    mh = [sum(w1s[k][i] * mean[i] for i in range(_C)) for k in range(_C)]
    w_eff = [[sum(w2s[c][k] * g[k] * w1s[k][j] for k in range(_C))
              for j in range(_C)] for c in range(_C)]
    b_eff = [p_ref[c, 2 * _C + 2]
             + sum(w2s[c][k] * (p_ref[k, 2 * _C + 1] - g[k] * mh[k])
                   for k in range(_C))
             for c in range(_C)]
    return w_eff, b_eff


def _apply_affine(xv, w_eff, b_eff):
    xs = [xv[:, c:c + 1, :, :] for c in range(_C)]
    return jnp.concatenate(
        [w_eff[c][0] * xs[0] + w_eff[c][1] * xs[1]
         + w_eff[c][2] * xs[2] + b_eff[c] for c in range(_C)], axis=1)


def _forward_resident(x4, params, inv_m, Nb):
    N, _, ROWS, _ = x4.shape
    T = N // Nb
    n_out = min(_OUT_SLOTS, T)

    def body(p_ref, x_hbm, o_hbm, xbuf, obuf, in_sems, out_sems):
        def in_copy(t):
            return pltpu.make_async_copy(
                x_hbm.at[pl.ds(t * Nb, Nb)], xbuf.at[pl.ds(t * Nb, Nb)],
                in_sems.at[t])

        def out_copy(t):
            return pltpu.make_async_copy(
                obuf.at[t % n_out], o_hbm.at[pl.ds(t * Nb, Nb)],
                out_sems.at[t % n_out])

        for t in range(T):
            in_copy(t).start()

        tot9 = None
        for t in range(T):
            in_copy(t).wait()
            parts = _tile_stats(xbuf[pl.ds(t * Nb, Nb)])
            tot9 = parts if tot9 is None else [
                a + b for a, b in zip(tot9, parts)]

        tot = [jnp.sum(v) for v in tot9]
        w_eff, b_eff = _fold_affine(tot, p_ref, inv_m)

        for t in range(T):
            if t >= n_out:
                out_copy(t - n_out).wait()
            obuf[t % n_out] = _apply_affine(
                xbuf[pl.ds(t * Nb, Nb)], w_eff, b_eff)
            out_copy(t).start()
        for t in range(max(0, T - n_out), T):
            out_copy(t).wait()

    return pl.pallas_call(
        body,
        out_shape=jax.ShapeDtypeStruct((N, _C, ROWS, _LANE), jnp.float32),
        in_specs=[pl.BlockSpec(memory_space=pltpu.MemorySpace.SMEM),
                  pl.BlockSpec(memory_space=pl.MemorySpace.ANY)],
        out_specs=pl.BlockSpec(memory_space=pl.MemorySpace.ANY),
        scratch_shapes=[
            pltpu.VMEM((N, _C, ROWS, _LANE), jnp.float32),
            pltpu.VMEM((_OUT_SLOTS, Nb, _C, ROWS, _LANE), jnp.float32),
            pltpu.SemaphoreType.DMA((N // Nb,)),
            pltpu.SemaphoreType.DMA((_OUT_SLOTS,)),
        ],
        compiler_params=pltpu.CompilerParams(
            vmem_limit_bytes=64 * 1024 * 1024),
    )(params, x4)


def _plan_tiles(rows, n):
    per_sample = _C * rows * _LANE * 4
    if per_sample <= _TILE_TARGET:
        s = rows
        nb = 1
        want = max(1, _TILE_TARGET // per_sample)
        for d in range(1, n + 1):
            if n % d == 0 and d <= want:
                nb = d
    else:
        nb = 1
        s = _SUB
        cap = _TILE_TARGET // (_C * _LANE * 4)
        for cand in range(_SUB, rows + 1, _SUB):
            if rows % cand == 0 and cand <= cap:
                s = cand
    return nb, s


def _forward_streaming(x4, params, inv_m):
    N, _, ROWS, _ = x4.shape
    Nb, S = _plan_tiles(ROWS, N)
    tn = N // Nb
    tr = ROWS // S
    T = tn * tr

    def body(p_ref, x_ref, o_ref, acc_ref, wb_ref):
        ph = pl.program_id(0)
        t = pl.program_id(1) * tr + pl.program_id(2)

        @pl.when(jnp.logical_and(ph == 0, t == 0))
        def _init():
            acc_ref[...] = jnp.zeros_like(acc_ref)

        @pl.when(ph == 0)
        def _stats():
            acc_ref[...] += jnp.stack(_tile_stats(x_ref[...]), axis=0)

        @pl.when(jnp.logical_and(ph == 0, t == T - 1))
        def _fold():
            tot = [jnp.sum(acc_ref[k]) for k in range(_NSTAT)]
            w_eff, b_eff = _fold_affine(tot, p_ref, inv_m)
            for c in range(_C):
                for j in range(_C):
                    wb_ref[c, j] = w_eff[c][j]
                wb_ref[c, _C] = b_eff[c]

        @pl.when(ph == 1)
        def _apply():
            w_eff = [[wb_ref[c, j] for j in range(_C)] for c in range(_C)]
            b_eff = [wb_ref[c, _C] for c in range(_C)]
            o_ref[...] = _apply_affine(x_ref[...], w_eff, b_eff)

    x_spec = pl.BlockSpec((Nb, _C, S, _LANE), lambda p, n, r: (n, 0, r, 0))
    o_spec = pl.BlockSpec(
        (Nb, _C, S, _LANE),
        lambda p, n, r: (jnp.where(p == 0, 0, n), 0,
                         jnp.where(p == 0, 0, r), 0))
    p_spec = pl.BlockSpec((_C, _NPARAM), lambda p, n, r: (0, 0),
                          memory_space=pltpu.MemorySpace.SMEM)

    return pl.pallas_call(
        body,
        out_shape=jax.ShapeDtypeStruct((N, _C, ROWS, _LANE), jnp.float32),
        grid=(2, tn, tr),
        in_specs=[p_spec, x_spec],
        out_specs=o_spec,
        scratch_shapes=[pltpu.VMEM((_NSTAT, _SUB, _LANE), jnp.float32),
                        pltpu.SMEM((_C, _C + 1), jnp.float32)],
        compiler_params=pltpu.CompilerParams(
            dimension_semantics=("arbitrary", "arbitrary", "arbitrary"),
            vmem_limit_bytes=64 * 1024 * 1024),
    )(params, x4)


def _fused_forward(x_nchw, w1, b1, gamma, beta, w2, b2):
    del b1
    N, c_in, H, W = x_nchw.shape
    assert c_in == _C
    HW = H * W
    inv_m = 1.0 / float(N * HW)

    HWp = _round_up(HW, _LANE * _SUB)
    ROWS = HWp // _LANE
    x3 = x_nchw.reshape(N, _C, HW).astype(jnp.float32)
    if HWp != HW:
        x3 = jnp.pad(x3, ((0, 0), (0, 0), (0, HWp - HW)))
    x4 = x3.reshape(N, _C, ROWS, _LANE)

    params = jnp.concatenate(
        [w1.astype(jnp.float32), w2.astype(jnp.float32),
         gamma.astype(jnp.float32)[:, None],
         beta.astype(jnp.float32)[:, None],
         b2.astype(jnp.float32)[:, None]], axis=1)

    per_sample = _C * ROWS * _LANE * 4
    if N * per_sample <= _RESIDENT_LIMIT:
        nb = 1
        want = max(1, _TILE_TARGET // per_sample)
        for d in range(1, N + 1):
            if N % d == 0 and d <= want:
                nb = d
        out4 = _forward_resident(x4, params, inv_m, nb)
    else:
        out4 = _forward_streaming(x4, params, inv_m)

    out3 = out4.reshape(N, _C, HWp)
    if HWp != HW:
        out3 = out3[:, :, :HW]
    return out3.reshape(N, _C, H, W)


def kernel(x_nchw, w1, b1, gamma, beta, w2, b2):
    return _fused_forward(x_nchw, w1, b1, gamma, beta, w2, b2)
